# baseline (speedup 1.0000x reference)
import functools

import jax
import jax.numpy as jnp
from jax.experimental import pallas as pl
from jax.experimental.pallas import tpu as pltpu
from jax.experimental.pallas import tpu_sc as plsc

N_POINTS = 2048
K = 16
R = 256
_GW = 128


def _dot(a, b):
    return jax.lax.dot(a, b, preferred_element_type=jnp.float32)


def _bdot(a, b):
    return jax.lax.dot(a.astype(jnp.bfloat16), b.astype(jnp.bfloat16),
                       preferred_element_type=jnp.float32)


def _knn_kernel(pts_ref, ptsT_ref, gidx_ref, *, rows):
    b = pl.program_id(0)
    pts_r = pts_ref[0]
    ptsT = ptsT_ref[0]
    sq_r = jnp.sum(pts_r * pts_r, axis=1, keepdims=True)
    sq_all = jnp.sum(ptsT * ptsT, axis=0, keepdims=True)
    d2 = sq_r + sq_all - 2.0 * _dot(pts_r, ptsT)
    lane_iota = jax.lax.broadcasted_iota(jnp.int32, (rows, N_POINTS), 1)
    for j in range(K):
        m = jnp.min(d2, axis=1, keepdims=True)
        amin = jnp.min(jnp.where(d2 == m, lane_iota, N_POINTS),
                       axis=1, keepdims=True)
        d2 = jnp.where(lane_iota == amin, jnp.float32(jnp.inf), d2)
        gidx_ref[0, :, j] = amin[:, 0] + b * N_POINTS


def _sc_gather(table, gidx_flat, n_rows, n_cols):
    mesh = plsc.VectorSubcoreMesh(core_axis_name="core",
                                  subcore_axis_name="subcore")

    @functools.partial(
        pl.kernel,
        out_type=jax.ShapeDtypeStruct((n_rows, n_cols), table.dtype),
        mesh=mesh,
        compiler_params=pltpu.CompilerParams(use_tc_tiling_on_sc=False),
    )
    def gather_kernel(x_hbm, i_hbm, o_hbm):
        def body(i_vmem, o_vmem):
            pltpu.sync_copy(x_hbm.at[i_vmem.at[0]], o_vmem)

        pltpu.emit_pipeline(
            body,
            grid=(n_rows // _GW,),
            in_specs=[pl.BlockSpec((1, _GW), index_map=lambda i: (0, i))],
            out_specs=[pl.BlockSpec((_GW, n_cols),
                                    index_map=lambda i: (i, 0))],
            core_axis_name=("core", "subcore"),
            dimension_semantics=(pltpu.PARALLEL,),
        )(i_hbm, o_hbm)

    return gather_kernel(table, gidx_flat.reshape(1, n_rows))


def _stage1_kernel(npos_ref, pts_ref, w1_ref, b1_ref, w2_ref, b2_ref,
                   f1_ref):
    npos = npos_ref[0, :, :, 0:3]
    rel = npos - pts_ref[0][:, None, :]
    rel_flat = rel.reshape(R * K, 3)
    h = jnp.maximum(_bdot(rel_flat, w1_ref[...]) + b1_ref[...], 0.0)
    h = jnp.maximum(_bdot(h, w2_ref[...]) + b2_ref[...], 0.0)
    f1_ref[0] = jnp.max(h.reshape(R, K, 128), axis=1)


def _stage2_kernel(npos_ref, pts_ref, nf_ref, w3_ref, b3_ref,
                   w4_ref, b4_ref, fgeo_ref, g_ref):
    t = pl.program_id(1)
    npos = npos_ref[0, :, :, 0:3]
    rel = npos - pts_ref[0][:, None, :]
    rel_flat = rel.reshape(R * K, 3)
    nf_bf = nf_ref[0].reshape(R * K, 128)
    w3r = w3_ref[0:3, :]
    w3f = w3_ref[3:, :]
    h = _bdot(rel_flat, w3r) + _bdot(nf_bf, w3f) + b3_ref[...]
    h = jnp.maximum(h, 0.0)
    h = jnp.maximum(_bdot(h, w4_ref[...]) + b4_ref[...], 0.0)
    fgeo = jnp.max(h.reshape(R, K, 256), axis=1)
    fgeo_ref[0] = fgeo

    @pl.when(t == 0)
    def _init():
        g_ref[...] = jnp.zeros_like(g_ref)

    g_ref[0, 0] += jnp.sum(fgeo, axis=0) / N_POINTS


R_KNN = 1024


def _knn(pts):
    B, N, _ = pts.shape
    ntiles = N // R_KNN
    ptsT = jnp.transpose(pts, (0, 2, 1))
    return pl.pallas_call(
        functools.partial(_knn_kernel, rows=R_KNN),
        grid=(B, ntiles),
        in_specs=[
            pl.BlockSpec((1, R_KNN, 3), lambda b, t: (b, t, 0)),
            pl.BlockSpec((1, 3, N), lambda b, t: (b, 0, 0)),
        ],
        out_specs=pl.BlockSpec((1, R_KNN, K), lambda b, t: (b, t, 0)),
        out_shape=jax.ShapeDtypeStruct((B, N, K), jnp.int32),
        compiler_params=pltpu.CompilerParams(
            dimension_semantics=("arbitrary", "parallel")),
    )(pts, ptsT)


def _pos_gather(pts, gidx):
    B, N, _ = pts.shape
    M = B * N * K
    pts8 = jnp.pad(pts.reshape(B * N, 3), ((0, 0), (0, 5)))
    return _sc_gather(pts8, gidx.reshape(M), M, 8).reshape(B, N, K, 8)


def _stage1(npos, pts, W1, b1, W2, b2):
    B, N, _ = pts.shape
    ntiles = N // R
    return pl.pallas_call(
        _stage1_kernel,
        grid=(B, ntiles),
        in_specs=[
            pl.BlockSpec((1, R, K, 8), lambda b, t: (b, t, 0, 0)),
            pl.BlockSpec((1, R, 3), lambda b, t: (b, t, 0)),
            pl.BlockSpec((3, 64), lambda b, t: (0, 0)),
            pl.BlockSpec((64,), lambda b, t: (0,)),
            pl.BlockSpec((64, 128), lambda b, t: (0, 0)),
            pl.BlockSpec((128,), lambda b, t: (0,)),
        ],
        out_specs=pl.BlockSpec((1, R, 128), lambda b, t: (b, t, 0)),
        out_shape=jax.ShapeDtypeStruct((B, N, 128), jnp.float32),
        compiler_params=pltpu.CompilerParams(
            dimension_semantics=("arbitrary", "parallel")),
    )(npos, pts, W1, b1, W2, b2)


def _f1_gather(f1p, gidx):
    B, N, _ = f1p.shape
    M = B * N * K
    nf = _sc_gather(f1p.reshape(B * N, 128), gidx.reshape(M), M, 128)
    return nf.reshape(B, N, K, 128)


def _stage2(npos, pts, nf, W3, b3, W4, b4):
    B, N, _ = pts.shape
    ntiles = N // R
    F_geo, g = pl.pallas_call(
        _stage2_kernel,
        grid=(B, ntiles),
        in_specs=[
            pl.BlockSpec((1, R, K, 8), lambda b, t: (b, t, 0, 0)),
            pl.BlockSpec((1, R, 3), lambda b, t: (b, t, 0)),
            pl.BlockSpec((1, R, K, 128), lambda b, t: (b, t, 0, 0)),
            pl.BlockSpec((3 + 128, 128), lambda b, t: (0, 0)),
            pl.BlockSpec((128,), lambda b, t: (0,)),
            pl.BlockSpec((128, 256), lambda b, t: (0, 0)),
            pl.BlockSpec((256,), lambda b, t: (0,)),
        ],
        out_specs=[
            pl.BlockSpec((1, R, 256), lambda b, t: (b, t, 0)),
            pl.BlockSpec((1, 1, 256), lambda b, t: (b, 0, 0)),
        ],
        out_shape=[
            jax.ShapeDtypeStruct((B, N, 256), jnp.float32),
            jax.ShapeDtypeStruct((B, 1, 256), jnp.float32),
        ],
    )(npos, pts, nf, W3, b3, W4, b4)

    return (F_geo, g[:, 0, :])


@jax.jit
def kernel(pts, W1, b1, W2, b2, W3, b3, W4, b4):
    halves = [pts[0:2], pts[2:4]]
    gidx = [_knn(h) for h in halves]
    npos = [_pos_gather(h, gi) for h, gi in zip(halves, gidx)]
    f1p = [_stage1(np_, h, W1, b1, W2, b2)
           for np_, h in zip(npos, halves)]
    nf = [_f1_gather(f, gi) for f, gi in zip(f1p, gidx)]
    outs = [_stage2(np_, h, nf_, W3, b3, W4, b4)
            for np_, h, nf_ in zip(npos, halves, nf)]
    F_geo = jnp.concatenate([o[0] for o in outs], axis=0)
    g = jnp.concatenate([o[1] for o in outs], axis=0)
    return (F_geo, g)


# speedup vs baseline: 1.1179x; 1.1179x over previous
import functools

import jax
import jax.numpy as jnp
from jax.experimental import pallas as pl
from jax.experimental.pallas import tpu as pltpu
from jax.experimental.pallas import tpu_sc as plsc

N_POINTS = 2048
K = 16
R = 512
_GW = 128


def _dot(a, b):
    return jax.lax.dot(a, b, preferred_element_type=jnp.float32)


def _bdot(a, b):
    return jax.lax.dot(a.astype(jnp.bfloat16), b.astype(jnp.bfloat16),
                       preferred_element_type=jnp.float32)


def _knn_kernel(pts_ref, ptsT_ref, gidx_ref, *, rows):
    b = pl.program_id(0)
    pts_r = pts_ref[0]
    ptsT = ptsT_ref[0]
    sq_r = jnp.sum(pts_r * pts_r, axis=1, keepdims=True)
    sq_all = jnp.sum(ptsT * ptsT, axis=0, keepdims=True)
    d2 = sq_r + sq_all - 2.0 * _dot(pts_r, ptsT)
    lane_iota = jax.lax.broadcasted_iota(jnp.int32, (rows, N_POINTS), 1)
    for j in range(K):
        m = jnp.min(d2, axis=1, keepdims=True)
        amin = jnp.min(jnp.where(d2 == m, lane_iota, N_POINTS),
                       axis=1, keepdims=True)
        d2 = jnp.where(lane_iota == amin, jnp.float32(jnp.inf), d2)
        gidx_ref[0, :, j] = amin[:, 0] + b * N_POINTS


def _sc_gather(table, gidx_flat, n_rows, n_cols):
    mesh = plsc.VectorSubcoreMesh(core_axis_name="core",
                                  subcore_axis_name="subcore")

    @functools.partial(
        pl.kernel,
        out_type=jax.ShapeDtypeStruct((n_rows, n_cols), table.dtype),
        mesh=mesh,
        compiler_params=pltpu.CompilerParams(use_tc_tiling_on_sc=False),
    )
    def gather_kernel(x_hbm, i_hbm, o_hbm):
        def body(i_vmem, o_vmem):
            pltpu.sync_copy(x_hbm.at[i_vmem.at[0]], o_vmem)

        pltpu.emit_pipeline(
            body,
            grid=(n_rows // _GW,),
            in_specs=[pl.BlockSpec((1, _GW), index_map=lambda i: (0, i))],
            out_specs=[pl.BlockSpec((_GW, n_cols),
                                    index_map=lambda i: (i, 0))],
            core_axis_name=("core", "subcore"),
            dimension_semantics=(pltpu.PARALLEL,),
        )(i_hbm, o_hbm)

    return gather_kernel(table, gidx_flat.reshape(1, n_rows))


def _stage1_kernel(npos_ref, pts_ref, w1_ref, b1_ref, w2_ref, b2_ref,
                   f1_ref):
    npos = npos_ref[0, :, :, 0:3]
    rel = npos - pts_ref[0][:, None, :]
    rel_flat = rel.reshape(R * K, 3)
    h = jnp.maximum(_bdot(rel_flat, w1_ref[...]) + b1_ref[...], 0.0)
    h = jnp.maximum(_bdot(h, w2_ref[...]) + b2_ref[...], 0.0)
    f1_ref[0] = jnp.max(h.reshape(R, K, 128), axis=1)


def _stage2_kernel(npos_ref, pts_ref, nf_ref, w3_ref, b3_ref,
                   w4_ref, b4_ref, fgeo_ref, g_ref):
    t = pl.program_id(1)
    npos = npos_ref[0, :, :, 0:3]
    rel = npos - pts_ref[0][:, None, :]
    rel_flat = rel.reshape(R * K, 3)
    nf_bf = nf_ref[0].reshape(R * K, 128)
    w3r = w3_ref[0:3, :]
    w3f = w3_ref[3:, :]
    h = _bdot(rel_flat, w3r) + _bdot(nf_bf, w3f) + b3_ref[...]
    h = jnp.maximum(h, 0.0)
    h = jnp.maximum(_bdot(h, w4_ref[...]) + b4_ref[...], 0.0)
    fgeo = jnp.max(h.reshape(R, K, 256), axis=1)
    fgeo_ref[0] = fgeo

    @pl.when(t == 0)
    def _init():
        g_ref[...] = jnp.zeros_like(g_ref)

    g_ref[0, 0] += jnp.sum(fgeo, axis=0) / N_POINTS


R_KNN = 512


def _knn(pts):
    B, N, _ = pts.shape
    ntiles = N // R_KNN
    ptsT = jnp.transpose(pts, (0, 2, 1))
    return pl.pallas_call(
        functools.partial(_knn_kernel, rows=R_KNN),
        grid=(B, ntiles),
        in_specs=[
            pl.BlockSpec((1, R_KNN, 3), lambda b, t: (b, t, 0)),
            pl.BlockSpec((1, 3, N), lambda b, t: (b, 0, 0)),
        ],
        out_specs=pl.BlockSpec((1, R_KNN, K), lambda b, t: (b, t, 0)),
        out_shape=jax.ShapeDtypeStruct((B, N, K), jnp.int32),
        compiler_params=pltpu.CompilerParams(
            dimension_semantics=("arbitrary", "parallel")),
    )(pts, ptsT)


def _pos_gather(pts, gidx):
    B, N, _ = pts.shape
    M = B * N * K
    pts8 = jnp.pad(pts.reshape(B * N, 3), ((0, 0), (0, 5)))
    return _sc_gather(pts8, gidx.reshape(M), M, 8).reshape(B, N, K, 8)


def _stage1(npos, pts, W1, b1, W2, b2):
    B, N, _ = pts.shape
    ntiles = N // R
    return pl.pallas_call(
        _stage1_kernel,
        grid=(B, ntiles),
        in_specs=[
            pl.BlockSpec((1, R, K, 8), lambda b, t: (b, t, 0, 0)),
            pl.BlockSpec((1, R, 3), lambda b, t: (b, t, 0)),
            pl.BlockSpec((3, 64), lambda b, t: (0, 0)),
            pl.BlockSpec((64,), lambda b, t: (0,)),
            pl.BlockSpec((64, 128), lambda b, t: (0, 0)),
            pl.BlockSpec((128,), lambda b, t: (0,)),
        ],
        out_specs=pl.BlockSpec((1, R, 128), lambda b, t: (b, t, 0)),
        out_shape=jax.ShapeDtypeStruct((B, N, 128), jnp.float32),
        compiler_params=pltpu.CompilerParams(
            dimension_semantics=("arbitrary", "parallel")),
    )(npos, pts, W1, b1, W2, b2)


def _f1_gather(f1p, gidx):
    B, N, _ = f1p.shape
    M = B * N * K
    nf = _sc_gather(f1p.reshape(B * N, 128), gidx.reshape(M), M, 128)
    return nf.reshape(B, N, K, 128)


def _stage2(npos, pts, nf, W3, b3, W4, b4):
    B, N, _ = pts.shape
    ntiles = N // R
    F_geo, g = pl.pallas_call(
        _stage2_kernel,
        grid=(B, ntiles),
        in_specs=[
            pl.BlockSpec((1, R, K, 8), lambda b, t: (b, t, 0, 0)),
            pl.BlockSpec((1, R, 3), lambda b, t: (b, t, 0)),
            pl.BlockSpec((1, R, K, 128), lambda b, t: (b, t, 0, 0)),
            pl.BlockSpec((3 + 128, 128), lambda b, t: (0, 0)),
            pl.BlockSpec((128,), lambda b, t: (0,)),
            pl.BlockSpec((128, 256), lambda b, t: (0, 0)),
            pl.BlockSpec((256,), lambda b, t: (0,)),
        ],
        out_specs=[
            pl.BlockSpec((1, R, 256), lambda b, t: (b, t, 0)),
            pl.BlockSpec((1, 1, 256), lambda b, t: (b, 0, 0)),
        ],
        out_shape=[
            jax.ShapeDtypeStruct((B, N, 256), jnp.float32),
            jax.ShapeDtypeStruct((B, 1, 256), jnp.float32),
        ],
    )(npos, pts, nf, W3, b3, W4, b4)

    return (F_geo, g[:, 0, :])


@jax.jit
def kernel(pts, W1, b1, W2, b2, W3, b3, W4, b4):
    halves = [pts[0:2], pts[2:4]]
    gidx = [_knn(h) for h in halves]
    npos = [_pos_gather(h, gi) for h, gi in zip(halves, gidx)]
    f1p = [_stage1(np_, h, W1, b1, W2, b2)
           for np_, h in zip(npos, halves)]
    nf = [_f1_gather(f, gi) for f, gi in zip(f1p, gidx)]
    outs = [_stage2(np_, h, nf_, W3, b3, W4, b4)
            for np_, h, nf_ in zip(npos, halves, nf)]
    F_geo = jnp.concatenate([o[0] for o in outs], axis=0)
    g = jnp.concatenate([o[1] for o in outs], axis=0)
    return (F_geo, g)


# speedup vs baseline: 1.1179x; 1.0000x over previous
import functools

import jax
import jax.numpy as jnp
from jax.experimental import pallas as pl
from jax.experimental.pallas import tpu as pltpu
from jax.experimental.pallas import tpu_sc as plsc

N_POINTS = 2048
K = 16
R = 512
_GW = 128


def _dot(a, b):
    return jax.lax.dot(a, b, preferred_element_type=jnp.float32)


def _bdot(a, b):
    return jax.lax.dot(a.astype(jnp.bfloat16), b.astype(jnp.bfloat16),
                       preferred_element_type=jnp.float32)


def _knn_kernel(pts_ref, ptsT_ref, gidx_ref, *, rows):
    b = pl.program_id(0)
    pts_r = pts_ref[0]
    ptsT = ptsT_ref[0]
    sq_r = jnp.sum(pts_r * pts_r, axis=1, keepdims=True)
    sq_all = jnp.sum(ptsT * ptsT, axis=0, keepdims=True)
    d2 = sq_r + sq_all - 2.0 * _dot(pts_r, ptsT)
    lane_iota = jax.lax.broadcasted_iota(jnp.int32, (rows, N_POINTS), 1)
    for j in range(K):
        m = jnp.min(d2, axis=1, keepdims=True)
        amin = jnp.min(jnp.where(d2 == m, lane_iota, N_POINTS),
                       axis=1, keepdims=True)
        d2 = jnp.where(lane_iota == amin, jnp.float32(jnp.inf), d2)
        gidx_ref[0, :, j] = amin[:, 0] + b * N_POINTS


def _sc_gather(table, gidx_flat, n_rows, n_cols):
    mesh = plsc.VectorSubcoreMesh(core_axis_name="core",
                                  subcore_axis_name="subcore")

    @functools.partial(
        pl.kernel,
        out_type=jax.ShapeDtypeStruct((n_rows, n_cols), table.dtype),
        mesh=mesh,
        compiler_params=pltpu.CompilerParams(use_tc_tiling_on_sc=False),
    )
    def gather_kernel(x_hbm, i_hbm, o_hbm):
        def body(i_vmem, o_vmem):
            pltpu.sync_copy(x_hbm.at[i_vmem.at[0]], o_vmem)

        pltpu.emit_pipeline(
            body,
            grid=(n_rows // _GW,),
            in_specs=[pl.BlockSpec((1, _GW), index_map=lambda i: (0, i))],
            out_specs=[pl.BlockSpec((_GW, n_cols),
                                    index_map=lambda i: (i, 0))],
            core_axis_name=("core", "subcore"),
            dimension_semantics=(pltpu.PARALLEL,),
        )(i_hbm, o_hbm)

    return gather_kernel(table, gidx_flat.reshape(1, n_rows))


def _stage1_kernel(npos_ref, pts_ref, w1_ref, b1_ref, w2_ref, b2_ref,
                   f1_ref):
    npos = npos_ref[0, :, :, 0:3]
    rel = npos - pts_ref[0][:, None, :]
    rel_flat = rel.reshape(R * K, 3)
    h = jnp.maximum(_bdot(rel_flat, w1_ref[...]) + b1_ref[...], 0.0)
    h = jnp.maximum(_bdot(h, w2_ref[...]) + b2_ref[...], 0.0)
    f1_ref[0] = jnp.max(h.reshape(R, K, 128), axis=1)


def _stage2_kernel(npos_ref, pts_ref, nf_ref, w3_ref, b3_ref,
                   w4_ref, b4_ref, fgeo_ref, g_ref):
    t = pl.program_id(1)
    npos = npos_ref[0, :, :, 0:3]
    rel = npos - pts_ref[0][:, None, :]
    rel_flat = rel.reshape(R * K, 3)
    nf_flat = nf_ref[0].reshape(R * K, 128)
    w3r = w3_ref[0:3, :]
    w3f = w3_ref[3:, :]
    h = _bdot(rel_flat, w3r) + _bdot(nf_flat, w3f) + b3_ref[...]
    h = jnp.maximum(h, 0.0)
    h = jnp.maximum(_bdot(h, w4_ref[...]) + b4_ref[...], 0.0)
    fgeo = jnp.max(h.reshape(R, K, 256), axis=1)
    fgeo_ref[0] = fgeo

    @pl.when(t == 0)
    def _init():
        g_ref[...] = jnp.zeros_like(g_ref)

    g_ref[0, 0] += jnp.sum(fgeo, axis=0) / N_POINTS


R_KNN = 512


def _knn(pts):
    B, N, _ = pts.shape
    ntiles = N // R_KNN
    ptsT = jnp.transpose(pts, (0, 2, 1))
    return pl.pallas_call(
        functools.partial(_knn_kernel, rows=R_KNN),
        grid=(B, ntiles),
        in_specs=[
            pl.BlockSpec((1, R_KNN, 3), lambda b, t: (b, t, 0)),
            pl.BlockSpec((1, 3, N), lambda b, t: (b, 0, 0)),
        ],
        out_specs=pl.BlockSpec((1, R_KNN, K), lambda b, t: (b, t, 0)),
        out_shape=jax.ShapeDtypeStruct((B, N, K), jnp.int32),
        compiler_params=pltpu.CompilerParams(
            dimension_semantics=("arbitrary", "parallel")),
    )(pts, ptsT)


def _pos_gather(pts, gidx):
    B, N, _ = pts.shape
    M = B * N * K
    pts8 = jnp.pad(pts.reshape(B * N, 3), ((0, 0), (0, 5)))
    return _sc_gather(pts8, gidx.reshape(M), M, 8).reshape(B, N, K, 8)


def _stage1(npos, pts, W1, b1, W2, b2):
    B, N, _ = pts.shape
    ntiles = N // R
    return pl.pallas_call(
        _stage1_kernel,
        grid=(B, ntiles),
        in_specs=[
            pl.BlockSpec((1, R, K, 8), lambda b, t: (b, t, 0, 0)),
            pl.BlockSpec((1, R, 3), lambda b, t: (b, t, 0)),
            pl.BlockSpec((3, 64), lambda b, t: (0, 0)),
            pl.BlockSpec((64,), lambda b, t: (0,)),
            pl.BlockSpec((64, 128), lambda b, t: (0, 0)),
            pl.BlockSpec((128,), lambda b, t: (0,)),
        ],
        out_specs=pl.BlockSpec((1, R, 128), lambda b, t: (b, t, 0)),
        out_shape=jax.ShapeDtypeStruct((B, N, 128), jnp.float32),
        compiler_params=pltpu.CompilerParams(
            dimension_semantics=("arbitrary", "parallel")),
    )(npos, pts, W1, b1, W2, b2)


def _f1_gather(f1p, gidx):
    B, N, _ = f1p.shape
    M = B * N * K
    nf = _sc_gather(f1p.reshape(B * N, 128), gidx.reshape(M), M, 128)
    return nf.reshape(B, N, K, 128)


def _stage2(npos, pts, nf, W3, b3, W4, b4):
    B, N, _ = pts.shape
    ntiles = N // R
    F_geo, g = pl.pallas_call(
        _stage2_kernel,
        grid=(B, ntiles),
        in_specs=[
            pl.BlockSpec((1, R, K, 8), lambda b, t: (b, t, 0, 0)),
            pl.BlockSpec((1, R, 3), lambda b, t: (b, t, 0)),
            pl.BlockSpec((1, R, K, 128), lambda b, t: (b, t, 0, 0)),
            pl.BlockSpec((3 + 128, 128), lambda b, t: (0, 0)),
            pl.BlockSpec((128,), lambda b, t: (0,)),
            pl.BlockSpec((128, 256), lambda b, t: (0, 0)),
            pl.BlockSpec((256,), lambda b, t: (0,)),
        ],
        out_specs=[
            pl.BlockSpec((1, R, 256), lambda b, t: (b, t, 0)),
            pl.BlockSpec((1, 1, 256), lambda b, t: (b, 0, 0)),
        ],
        out_shape=[
            jax.ShapeDtypeStruct((B, N, 256), jnp.float32),
            jax.ShapeDtypeStruct((B, 1, 256), jnp.float32),
        ],
    )(npos, pts, nf, W3, b3, W4, b4)

    return (F_geo, g[:, 0, :])


@jax.jit
def kernel(pts, W1, b1, W2, b2, W3, b3, W4, b4):
    B = pts.shape[0]
    halves = [pts[0:B // 2], pts[B // 2:]]
    gidx = [_knn(h) for h in halves]
    npos = [_pos_gather(h, gi) for h, gi in zip(halves, gidx)]
    f1p = [_stage1(np_, h, W1, b1, W2, b2)
           for np_, h in zip(npos, halves)]
    nf = [_f1_gather(f, gi) for f, gi in zip(f1p, gidx)]
    outs = [_stage2(np_, h, nf_, W3, b3, W4, b4)
            for np_, h, nf_ in zip(npos, halves, nf)]
    F_geo = jnp.concatenate([o[0] for o in outs], axis=0)
    g = jnp.concatenate([o[1] for o in outs], axis=0)
    return (F_geo, g)
